# Initial kernel scaffold; baseline (speedup 1.0000x reference)
import jax
import jax.numpy as jnp
from jax.experimental import pallas as pl


def kernel(x, pos, edge_index, params):
    raise NotImplementedError("write your pallas kernel here")


# speedup vs baseline: 5.5564x; 5.5564x over previous
import functools

import jax
import jax.numpy as jnp
from jax import lax
from jax.experimental import pallas as pl
from jax.experimental.pallas import tpu as pltpu
from jax.experimental.pallas import tpu_sc as plsc

NC, NS = 2, 16
NW = NC * NS

_N = 50000
_E = 800000
_CH = [1, 16, 64, 128, 256, 256]
_POOL_M = [12500, 3125, 781, 195]
_POOL_E = [200000, 50000, 12500, 3125]
_POOL_SIZES = [(5.0, 5.0, 10.0), (2.0, 2.0, 1.0), (2.0, 2.0, 1.0), (1.0, 1.0, 1.0)]

_LN = [_N] + _POOL_M
_LC = _CH[1:]
_LE = [_E] + _POOL_E

_ICHUNK = 128


def _ceil(a, b):
    return -(-a // b) * b


_NP = [max(256, _ceil(n + 1, 256)) for n in _LN]
_EW = [_ceil(-(-e // NW), _ICHUNK) for e in _LE]
_EFW = [_ceil(-(-(2 * e) // NW), 16) for e in _POOL_E]


def _row_chunk(nw):
    best = 8
    for d in range(8, _ICHUNK + 1, 8):
        if nw % d == 0:
            best = d
    return best


def _make_mp(lvl):
    n_pad, C, ew = _NP[lvl], _LC[lvl], _EW[lvl]
    zr = n_pad // NS
    mesh = plsc.VectorSubcoreMesh(core_axis_name="c", subcore_axis_name="s", num_cores=NC, num_subcores=NS)

    @functools.partial(
        pl.kernel,
        out_type=(
            jax.ShapeDtypeStruct((NC, n_pad, C), jnp.float32),
            jax.ShapeDtypeStruct((NC, n_pad, 16), jnp.float32),
        ),
        mesh=mesh,
        scratch_types=[
            pltpu.VMEM((_ICHUNK,), jnp.int32),
            pltpu.VMEM((_ICHUNK,), jnp.int32),
            pltpu.VMEM((_ICHUNK, C), jnp.float32),
            pltpu.VMEM((_ICHUNK, 16), jnp.float32),
            pltpu.VMEM_SHARED((n_pad, C), jnp.float32),
            pltpu.VMEM_SHARED((n_pad, 16), jnp.float32),
            pltpu.SemaphoreType.DMA,
        ],
        compiler_params=pltpu.CompilerParams(use_tc_tiling_on_sc=False, needs_layout_passes=False),
        name=f"sc_mp_l{lvl}",
    )
    def mp(h_hbm, src_hbm, dst_hbm, ones_hbm, zc_hbm, z16_hbm,
           agg_out, deg_out, sidx, didx, rows, ones_v, agg_sh, deg_sh, sem):
        cid = lax.axis_index("c")
        sid = lax.axis_index("s")
        wid = sid * NC + cid
        r0 = sid * zr
        pltpu.sync_copy(zc_hbm.at[pl.ds(r0, zr)], agg_sh.at[pl.ds(r0, zr)])
        pltpu.sync_copy(z16_hbm.at[pl.ds(r0, zr)], deg_sh.at[pl.ds(r0, zr)])
        pltpu.sync_copy(ones_hbm, ones_v)
        plsc.subcore_barrier()
        e0 = wid * ew

        @pl.loop(0, ew // _ICHUNK)
        def _(i):
            b = pl.multiple_of(e0 + i * _ICHUNK, _ICHUNK)
            pltpu.sync_copy(src_hbm.at[pl.ds(b, _ICHUNK)], sidx)
            pltpu.sync_copy(dst_hbm.at[pl.ds(b, _ICHUNK)], didx)
            pltpu.async_copy(h_hbm.at[sidx], rows, sem).wait()
            pltpu.sync_copy(rows, agg_sh.at[didx], add=True)
            pltpu.sync_copy(ones_v, deg_sh.at[didx], add=True)

        plsc.subcore_barrier()
        pltpu.sync_copy(agg_sh.at[pl.ds(r0, zr)], agg_out.at[cid, pl.ds(r0, zr)])
        pltpu.sync_copy(deg_sh.at[pl.ds(r0, zr)], deg_out.at[cid, pl.ds(r0, zr)])

    return mp


def _make_pool(i):
    n_pad, m_pad = _NP[i], _NP[i + 1]
    cc = _LC[i] + 32
    nw = n_pad // NW
    rc = _row_chunk(nw)
    efw = _EFW[i]
    zrm = m_pad // NS
    mesh = plsc.VectorSubcoreMesh(core_axis_name="c", subcore_axis_name="s", num_cores=NC, num_subcores=NS)

    @functools.partial(
        pl.kernel,
        out_type=(
            jax.ShapeDtypeStruct((NC, m_pad, cc), jnp.float32),
            jax.ShapeDtypeStruct((NW * efw,), jnp.int32),
        ),
        mesh=mesh,
        scratch_types=[
            pltpu.VMEM((rc,), jnp.int32),
            pltpu.VMEM((rc, cc), jnp.float32),
            pltpu.VMEM((n_pad,), jnp.int32),
            pltpu.VMEM((efw,), jnp.int32),
            pltpu.VMEM((efw,), jnp.int32),
            pltpu.VMEM_SHARED((m_pad, cc), jnp.float32),
        ],
        compiler_params=pltpu.CompilerParams(use_tc_tiling_on_sc=False, needs_layout_passes=False),
        name=f"sc_pool_{i}",
    )
    def pool(xcat_hbm, cl_hbm, ef_hbm, zm_hbm,
             sums_out, ne_out, cidx, rows, table, eb, ob, sums_sh):
        cid = lax.axis_index("c")
        sid = lax.axis_index("s")
        wid = sid * NC + cid
        r0m = sid * zrm
        pltpu.sync_copy(zm_hbm.at[pl.ds(r0m, zrm)], sums_sh.at[pl.ds(r0m, zrm)])
        plsc.subcore_barrier()
        row0 = wid * nw

        @pl.loop(0, nw // rc)
        def _(j):
            b = pl.multiple_of(row0 + j * rc, rc)
            pltpu.sync_copy(cl_hbm.at[pl.ds(b, rc)], cidx)
            pltpu.sync_copy(xcat_hbm.at[pl.ds(b, rc)], rows)
            pltpu.sync_copy(rows, sums_sh.at[cidx], add=True)

        pltpu.sync_copy(cl_hbm, table)
        pltpu.sync_copy(ef_hbm.at[pl.ds(wid * efw, efw)], eb)

        @pl.loop(0, efw // 16)
        def _(t):
            idx = eb[pl.ds(t * 16, 16)]
            ob[pl.ds(t * 16, 16)] = plsc.load_gather(table, [idx])

        pltpu.sync_copy(ob, ne_out.at[pl.ds(wid * efw, efw)])
        plsc.subcore_barrier()
        pltpu.sync_copy(sums_sh.at[pl.ds(r0m, zrm)],
                        sums_out.at[cid, pl.ds(r0m, zrm)])

    return pool


_NB = 16


def _make_tc_a0():
    n_pad = _NP[0]
    br = n_pad // _NB

    def body(x_ref, w1_ref, b1_ref, h_ref):
        h_ref[...] = x_ref[...] * w1_ref[...] + b1_ref[...]

    return pl.pallas_call(
        body,
        grid=(_NB,),
        in_specs=[
            pl.BlockSpec((br, 1), lambda b: (b, 0)),
            pl.BlockSpec((1, _LC[0]), lambda b: (0, 0)),
            pl.BlockSpec((1, _LC[0]), lambda b: (0, 0)),
        ],
        out_specs=pl.BlockSpec((br, _LC[0]), lambda b: (b, 0)),
        out_shape=jax.ShapeDtypeStruct((n_pad, _LC[0]), jnp.float32),
        name="tc_a0",
    )


def _make_tc_diva(i):
    c_in = _LC[i]
    c_out = _LC[i + 1]
    m_pad = _NP[i + 1]

    br = m_pad // _NB

    def body(s0_ref, s1_ref, w1_ref, b1_ref, h_ref, pos_ref):
        s = s0_ref[...] + s1_ref[...]
        den = jnp.maximum(s[:, c_in + 16:c_in + 17], 1.0)
        xn = s[:, :c_in] / den
        pos_ref[...] = s[:, c_in:c_in + 16] / den
        h_ref[...] = (
            jnp.dot(xn, w1_ref[...], preferred_element_type=jnp.float32)
            + b1_ref[...]
        )

    cc = c_in + 32
    return pl.pallas_call(
        body,
        grid=(_NB,),
        in_specs=[
            pl.BlockSpec((br, cc), lambda b: (b, 0)),
            pl.BlockSpec((br, cc), lambda b: (b, 0)),
            pl.BlockSpec((c_in, c_out), lambda b: (0, 0)),
            pl.BlockSpec((1, c_out), lambda b: (0, 0)),
        ],
        out_specs=(
            pl.BlockSpec((br, c_out), lambda b: (b, 0)),
            pl.BlockSpec((br, 16), lambda b: (b, 0)),
        ),
        out_shape=(
            jax.ShapeDtypeStruct((m_pad, c_out), jnp.float32),
            jax.ShapeDtypeStruct((m_pad, 16), jnp.float32),
        ),
        name=f"tc_diva_{i}",
    )


def _make_tc_b(lvl):
    n, c = _LN[lvl], _LC[lvl]
    n_pad = _NP[lvl]
    last = lvl == 4
    if not last:
        vox = _POOL_SIZES[lvl]
        m = _POOL_M[lvl]
        mp_pad = _NP[lvl + 1]

    br = n_pad // _NB

    def body(h_ref, a0_ref, a1_ref, d0_ref, d1_ref, pos_ref, w2_ref, b2_ref,
             g_ref, be_ref, *refs):
        out_refs, stats = refs[:-1], refs[-1]
        p = pl.program_id(0)
        b = pl.program_id(1)

        @pl.when((p == 0) & (b == 0))
        def _():
            stats[...] = jnp.zeros_like(stats)

        agg = a0_ref[...] + a1_ref[...]
        deg = d0_ref[...] + d1_ref[...]
        den = jnp.maximum(deg[:, 0:1], 1.0)
        z = h_ref[...] + agg / den
        out = (
            jnp.dot(z, w2_ref[...], preferred_element_type=jnp.float32)
            + b2_ref[...]
        )
        rid = b * br + lax.broadcasted_iota(jnp.int32, (br, 1), 0)
        msk = rid < n

        @pl.when(p == 0)
        def _():
            om = jnp.where(msk, out, 0.0)
            stats[0:1, :] += jnp.sum(om, axis=0, keepdims=True)

        @pl.when(p == 1)
        def _():
            mu = stats[0:1, :] / n
            dlt = jnp.where(msk, out - mu, 0.0)
            stats[1:2, :] += jnp.sum(dlt * dlt, axis=0, keepdims=True)

        @pl.when(p == 2)
        def _():
            mu = stats[0:1, :] / n
            var = stats[1:2, :] / n
            y = g_ref[...] * (out - mu) / jnp.sqrt(var + 1e-5) + be_ref[...]
            xn = jnp.where(msk, jnp.maximum(y, 0.0), 0.0)
            if last:
                out_refs[0][...] = xn
            else:
                pos = pos_ref[...]
                v0 = jnp.floor(pos[:, 0:1] / vox[0]).astype(jnp.int32)
                v1 = jnp.floor(pos[:, 1:2] / vox[1]).astype(jnp.int32)
                v2 = jnp.floor(pos[:, 2:3] / vox[2]).astype(jnp.int32)
                hh = (v0 * 73856093) ^ (v1 * 19349663) ^ (v2 * 83492791)
                cl = jnp.where(msk, jnp.mod(hh, m),
                               m + jnp.mod(rid, mp_pad - m))
                ones = jnp.ones((br, 16), jnp.float32)
                out_refs[0][...] = jnp.concatenate([xn, pos, ones], axis=1)
                out_refs[1][...] = cl

    rb = lambda w: pl.BlockSpec((br, w), lambda p, b: (b, 0))
    cb = lambda r, w: pl.BlockSpec((r, w), lambda p, b: (0, 0))
    in_specs = [rb(c), rb(c), rb(c), rb(16), rb(16), rb(16),
                cb(c, c), cb(1, c), cb(1, c), cb(1, c)]
    if last:
        out_shape = (jax.ShapeDtypeStruct((n_pad, c), jnp.float32),)
        out_specs = (rb(c),)
    else:
        out_shape = (
            jax.ShapeDtypeStruct((n_pad, c + 32), jnp.float32),
            jax.ShapeDtypeStruct((n_pad, 1), jnp.int32),
        )
        out_specs = (rb(c + 32), rb(1))
    return pl.pallas_call(
        body,
        grid=(3, _NB),
        in_specs=in_specs,
        out_specs=out_specs,
        out_shape=out_shape,
        scratch_shapes=[pltpu.VMEM((8, c), jnp.float32)],
        name=f"tc_b_{lvl}",
    )


_MP = [_make_mp(l) for l in range(5)]
_POOL = [_make_pool(i) for i in range(4)]
_TC_A0 = _make_tc_a0()
_TC_DIVA = [_make_tc_diva(i) for i in range(4)]
_TC_B = [_make_tc_b(l) for l in range(5)]


def _tc_b(lvl, h, agg2, deg2, pos, p):
    return _TC_B[lvl](h, agg2[0], agg2[1], deg2[0], deg2[1], pos,
                      p["W2"], p["b2"].reshape(1, -1),
                      p["gamma"].reshape(1, -1), p["beta"].reshape(1, -1))


def _mp_edges(edges, lvl):
    e = edges.shape[1]
    n, n_pad = _LN[lvl], _NP[lvl]
    ep = NW * _EW[lvl]
    pad = jnp.arange(ep - e, dtype=jnp.int32)
    src = jnp.concatenate([edges[0], pad % n])
    dst = jnp.concatenate([edges[1], n + pad % (n_pad - n)])
    return src, dst


def _run_mp(h, edges, lvl):
    n_pad, c = _NP[lvl], _LC[lvl]
    src, dst = _mp_edges(edges, lvl)
    ones = jnp.ones((_ICHUNK, 16), jnp.float32)
    zc = jnp.zeros((n_pad, c), jnp.float32)
    z16 = jnp.zeros((n_pad, 16), jnp.float32)
    return _MP[lvl](h, src, dst, ones, zc, z16)


def kernel(x, pos, edge_index, params):
    edges = edge_index.astype(jnp.int32)
    n_pad0 = _NP[0]
    x_p = jnp.pad(x, ((0, n_pad0 - _N), (0, 0)))
    pos_p = jnp.pad(pos, ((0, n_pad0 - _N), (0, 13)))

    p = params[0]
    h = _TC_A0(x_p, p["W1"], p["b1"].reshape(1, -1))
    agg2, deg2 = _run_mp(h, edges, 0)
    xcat, cl = _tc_b(0, h, agg2, deg2, pos_p, p)

    for i in range(4):
        ek = _POOL_E[i]
        efp = NW * _EFW[i]
        ef = jnp.concatenate([
            edges[:, :ek].reshape(-1),
            jnp.zeros((efp - 2 * ek,), jnp.int32),
        ])
        zm = jnp.zeros((_NP[i + 1], _LC[i] + 32), jnp.float32)
        sums2, ne = _POOL[i](xcat, cl.reshape(-1), ef, zm)
        edges = ne[:2 * ek].reshape(2, ek)

        p = params[i + 1]
        h, posn = _TC_DIVA[i](sums2[0], sums2[1], p["W1"],
                              p["b1"].reshape(1, -1))
        agg2, deg2 = _run_mp(h, edges, i + 1)
        if i < 3:
            xcat, cl = _tc_b(i + 1, h, agg2, deg2, posn, p)
        else:
            xfin, = _tc_b(4, h, agg2, deg2, posn, p)
    return xfin[:_LN[4]]
